# Initial kernel scaffold; baseline (speedup 1.0000x reference)
import jax
import jax.numpy as jnp
from jax.experimental import pallas as pl


def kernel(input_ids, W):
    raise NotImplementedError("write your pallas kernel here")


# speedup vs baseline: 1.0011x; 1.0011x over previous
import functools

import jax
import jax.numpy as jnp
from jax import lax
from jax.experimental import pallas as pl
from jax.experimental.pallas import tpu as pltpu
from jax.experimental.pallas import tpu_sc as plsc

VOCAB = 32000
HIDDEN = 2048
BATCH = 4
SEQ = 4096

NUM_CORES = 2
NUM_SUBCORES = 16
NUM_WORKERS = NUM_CORES * NUM_SUBCORES
ROWS = BATCH * SEQ
ROWS_PER_W = ROWS // NUM_WORKERS
CHUNK = 32
N_CHUNKS = ROWS_PER_W // CHUNK


def _make_gather():
  mesh = plsc.VectorSubcoreMesh(
      core_axis_name="c", subcore_axis_name="s",
      num_cores=NUM_CORES, num_subcores=NUM_SUBCORES)

  @functools.partial(
      pl.kernel,
      out_type=jax.ShapeDtypeStruct((ROWS, HIDDEN), jnp.float32),
      mesh=mesh,
      scratch_types=[
          pltpu.VMEM((ROWS_PER_W,), jnp.int32),
          pltpu.VMEM((CHUNK, HIDDEN), jnp.float32),
          pltpu.SemaphoreType.DMA,
      ],
  )
  def gather_kernel(idx_hbm, table_hbm, out_hbm, idx_v, buf, sem):
    wid = lax.axis_index("s") * NUM_CORES + lax.axis_index("c")
    base = wid * ROWS_PER_W
    pltpu.sync_copy(idx_hbm.at[pl.ds(base, ROWS_PER_W)], idx_v)

    def body(i, carry):
      off = pl.multiple_of(i * CHUNK, CHUNK)
      pltpu.async_copy(
          table_hbm.at[idx_v.at[pl.ds(off, CHUNK)]], buf, sem).wait()
      pltpu.sync_copy(buf, out_hbm.at[pl.ds(base + off, CHUNK)])
      return carry

    lax.fori_loop(0, N_CHUNKS, body, 0)

  return gather_kernel


_gather = _make_gather()


def kernel(input_ids, W):
  idx = input_ids.T.reshape(-1).astype(jnp.int32)
  out = _gather(idx, W)
  return out.reshape(SEQ, BATCH, HIDDEN)


# speedup vs baseline: 1.0312x; 1.0300x over previous
import functools

import jax
import jax.numpy as jnp
from jax import lax
from jax.experimental import pallas as pl
from jax.experimental.pallas import tpu as pltpu
from jax.experimental.pallas import tpu_sc as plsc

VOCAB = 32000
HIDDEN = 2048
BATCH = 4
SEQ = 4096

NUM_CORES = 2
NUM_SUBCORES = 16
NUM_WORKERS = NUM_CORES * NUM_SUBCORES
ROWS = BATCH * SEQ
ROWS_PER_W = ROWS // NUM_WORKERS
CHUNK = 16
N_CHUNKS = ROWS_PER_W // CHUNK
NBUF = 2


def _make_gather():
  mesh = plsc.VectorSubcoreMesh(
      core_axis_name="c", subcore_axis_name="s",
      num_cores=NUM_CORES, num_subcores=NUM_SUBCORES)

  @functools.partial(
      pl.kernel,
      out_type=jax.ShapeDtypeStruct((ROWS, HIDDEN), jnp.float32),
      mesh=mesh,
      scratch_types=[
          pltpu.VMEM((ROWS_PER_W,), jnp.int32),
          [pltpu.VMEM((CHUNK, HIDDEN), jnp.float32) for _ in range(NBUF)],
          [pltpu.SemaphoreType.DMA for _ in range(NBUF)],
          [pltpu.SemaphoreType.DMA for _ in range(NBUF)],
      ],
  )
  def gather_kernel(idx_hbm, table_hbm, out_hbm, idx_v, bufs, gsems, wsems):
    wid = lax.axis_index("s") * NUM_CORES + lax.axis_index("c")
    base = wid * ROWS_PER_W
    pltpu.sync_copy(idx_hbm.at[pl.ds(base, ROWS_PER_W)], idx_v)

    def gather_chunk(j, b):
      off = pl.multiple_of(j * CHUNK, CHUNK)
      return pltpu.make_async_copy(
          table_hbm.at[idx_v.at[pl.ds(off, CHUNK)]], bufs[b], gsems[b])

    def write_chunk(j, b):
      off = pl.multiple_of(j * CHUNK, CHUNK)
      return pltpu.make_async_copy(
          bufs[b], out_hbm.at[pl.ds(base + off, CHUNK)], wsems[b])

    for b in range(NBUF):
      gather_chunk(b, b).start()

    def body(p, carry):
      for b in range(NBUF):
        j = p * NBUF + b
        gather_chunk(j, b).wait()
        write_chunk(j, b).start()
        write_chunk(j, b).wait()
        gather_chunk(j + NBUF, b).start()
      return carry

    lax.fori_loop(0, N_CHUNKS // NBUF - 1, body, 0)

    for b in range(NBUF):
      j = N_CHUNKS - NBUF + b
      gather_chunk(j, b).wait()
      write_chunk(j, b).start()
    for b in range(NBUF):
      write_chunk(N_CHUNKS - NBUF + b, b).wait()

  return gather_kernel


_gather = _make_gather()


def kernel(input_ids, W):
  idx = input_ids.T.reshape(-1).astype(jnp.int32)
  out = _gather(idx, W)
  return out.reshape(SEQ, BATCH, HIDDEN)


# speedup vs baseline: 2.3041x; 2.2344x over previous
import functools

import jax
import jax.numpy as jnp
from jax import lax
from jax.experimental import pallas as pl
from jax.experimental.pallas import tpu as pltpu
from jax.experimental.pallas import tpu_sc as plsc

VOCAB = 32000
HIDDEN = 2048
BATCH = 4
SEQ = 4096

NUM_CORES = 2
NUM_SUBCORES = 16
NUM_WORKERS = NUM_CORES * NUM_SUBCORES
ROWS = BATCH * SEQ
ROWS_PER_W = ROWS // NUM_WORKERS
CHUNK = 16
N_CHUNKS = ROWS_PER_W // CHUNK
NBUF = 2


def _make_gather():
  mesh = plsc.VectorSubcoreMesh(
      core_axis_name="c", subcore_axis_name="s",
      num_cores=NUM_CORES, num_subcores=NUM_SUBCORES)

  @functools.partial(
      pl.kernel,
      out_type=jax.ShapeDtypeStruct((SEQ, BATCH, HIDDEN), jnp.float32),
      mesh=mesh,
      scratch_types=[
          pltpu.VMEM((ROWS_PER_W,), jnp.int32),
          [pltpu.VMEM((CHUNK, HIDDEN), jnp.float32) for _ in range(NBUF)],
          [pltpu.SemaphoreType.DMA for _ in range(NBUF)],
          [pltpu.SemaphoreType.DMA for _ in range(NBUF)],
      ],
  )
  def gather_kernel(idx_hbm, table_hbm, out_hbm, idx_v, bufs, gsems, wsems):
    wid = lax.axis_index("s") * NUM_CORES + lax.axis_index("c")
    base = wid * ROWS_PER_W
    s_base = wid * (ROWS_PER_W // BATCH)
    pltpu.sync_copy(idx_hbm.at[pl.ds(base, ROWS_PER_W)], idx_v)

    def gather_chunk(j, b):
      off = pl.multiple_of(j * CHUNK, CHUNK)
      return pltpu.make_async_copy(
          table_hbm.at[idx_v.at[pl.ds(off, CHUNK)]], bufs[b], gsems[b])

    def write_descs(j, b):
      s_off = s_base + j * (CHUNK // BATCH)
      return [
          pltpu.make_async_copy(
              bufs[b].at[pl.ds(k * BATCH, BATCH)], out_hbm.at[s_off + k],
              wsems[b])
          for k in range(CHUNK // BATCH)
      ]

    def write_chunk_start(j, b):
      for cp in write_descs(j, b):
        cp.start()

    def write_chunk_wait(j, b):
      for cp in write_descs(j, b):
        cp.wait()

    for b in range(NBUF):
      gather_chunk(b, b).start()

    def body(p, carry):
      for b in range(NBUF):
        j = p * NBUF + b
        gather_chunk(j, b).wait()
        write_chunk_start(j, b)
        write_chunk_wait(j, b)
        gather_chunk(j + NBUF, b).start()
      return carry

    lax.fori_loop(0, N_CHUNKS // NBUF - 1, body, 0)

    for b in range(NBUF):
      j = N_CHUNKS - NBUF + b
      gather_chunk(j, b).wait()
      write_chunk_start(j, b)
    for b in range(NBUF):
      write_chunk_wait(N_CHUNKS - NBUF + b, b)

  return gather_kernel


_gather = _make_gather()


def kernel(input_ids, W):
  idx = input_ids.T.reshape(-1).astype(jnp.int32)
  return _gather(idx, W)


# speedup vs baseline: 2.3241x; 1.0086x over previous
import functools

import jax
import jax.numpy as jnp
from jax import lax
from jax.experimental import pallas as pl
from jax.experimental.pallas import tpu as pltpu
from jax.experimental.pallas import tpu_sc as plsc

VOCAB = 32000
HIDDEN = 2048
BATCH = 4
SEQ = 4096

NUM_CORES = 2
NUM_SUBCORES = 16
NUM_WORKERS = NUM_CORES * NUM_SUBCORES
ROWS = BATCH * SEQ
ROWS_PER_W = ROWS // NUM_WORKERS
CHUNK = 8
N_CHUNKS = ROWS_PER_W // CHUNK
NBUF = 4


def _make_gather():
  mesh = plsc.VectorSubcoreMesh(
      core_axis_name="c", subcore_axis_name="s",
      num_cores=NUM_CORES, num_subcores=NUM_SUBCORES)

  @functools.partial(
      pl.kernel,
      out_type=jax.ShapeDtypeStruct((SEQ, BATCH, HIDDEN), jnp.float32),
      mesh=mesh,
      scratch_types=[
          pltpu.VMEM((ROWS_PER_W,), jnp.int32),
          [pltpu.VMEM((CHUNK, HIDDEN), jnp.float32) for _ in range(NBUF)],
          [pltpu.SemaphoreType.DMA for _ in range(NBUF)],
          [pltpu.SemaphoreType.DMA for _ in range(NBUF)],
      ],
  )
  def gather_kernel(idx_hbm, table_hbm, out_hbm, idx_v, bufs, gsems, wsems):
    wid = lax.axis_index("s") * NUM_CORES + lax.axis_index("c")
    base = wid * ROWS_PER_W
    s_base = wid * (ROWS_PER_W // BATCH)
    pltpu.sync_copy(idx_hbm.at[pl.ds(base, ROWS_PER_W)], idx_v)

    def gather_chunk(j, b):
      off = pl.multiple_of(j * CHUNK, CHUNK)
      return pltpu.make_async_copy(
          table_hbm.at[idx_v.at[pl.ds(off, CHUNK)]], bufs[b], gsems[b])

    def write_descs(j, b):
      s_off = s_base + j * (CHUNK // BATCH)
      return [
          pltpu.make_async_copy(
              bufs[b].at[pl.ds(k * BATCH, BATCH)], out_hbm.at[s_off + k],
              wsems[b])
          for k in range(CHUNK // BATCH)
      ]

    def write_chunk_start(j, b):
      for cp in write_descs(j, b):
        cp.start()

    def write_chunk_wait(j, b):
      for cp in write_descs(j, b):
        cp.wait()

    for b in range(NBUF):
      gather_chunk(b, b).start()

    def body(p, carry):
      for b in range(NBUF):
        j = p * NBUF + b
        gather_chunk(j, b).wait()
        write_chunk_start(j, b)
        write_chunk_wait(j, b)
        gather_chunk(j + NBUF, b).start()
      return carry

    lax.fori_loop(0, N_CHUNKS // NBUF - 1, body, 0)

    for b in range(NBUF):
      j = N_CHUNKS - NBUF + b
      gather_chunk(j, b).wait()
      write_chunk_start(j, b)
    for b in range(NBUF):
      write_chunk_wait(N_CHUNKS - NBUF + b, b)

  return gather_kernel


_gather = _make_gather()


def kernel(input_ids, W):
  idx = input_ids.T.reshape(-1).astype(jnp.int32)
  return _gather(idx, W)
